# baseline (speedup 1.0000x reference)
import math

import jax
import jax.numpy as jnp
from jax.experimental import pallas as pl
from jax.experimental.pallas import tpu as pltpu
from jax.experimental.pallas import tpu_sc as plsc

EMB = 64
SCALE = math.sqrt(EMB)
CHUNKS = ((0, 0, 104), (0, 104, 96), (1, 0, 104), (1, 104, 96))
WINDOW = 400


def kernel(tokens, embedding_weight):
    n_rows, n_cols = tokens.shape
    n_tok = n_rows * n_cols
    idx = tokens.astype(jnp.int32)

    mesh = plsc.VectorSubcoreMesh(core_axis_name="core", subcore_axis_name="subcore")

    @jax.jit
    def run(table, indices):
        @pl.kernel(
            out_type=jax.ShapeDtypeStruct((n_tok, EMB), jnp.float32),
            mesh=mesh,
            scratch_types=[pltpu.SemaphoreType.DMA((len(CHUNKS),))],
            compiler_params=pltpu.CompilerParams(use_tc_tiling_on_sc=False),
        )
        def k(x_hbm, i_hbm, o_hbm, sems):
            def body(i_vmem, o_vmem):
                cps = []
                base = 0
                bases = []
                for j, (row, col, cnt) in enumerate(CHUNKS):
                    cps.append(
                        pltpu.async_copy(
                            x_hbm.at[i_vmem.at[row, pl.ds(col, cnt)]],
                            o_vmem.at[pl.ds(base, cnt)],
                            sems.at[j],
                        )
                    )
                    bases.append(base)
                    base += cnt
                for j, (row, col, cnt) in enumerate(CHUNKS):
                    cps[j].wait()

                    @pl.loop(0, cnt)
                    def _(r, j=j, base=bases[j]):
                        vrow = o_vmem.at[base + r]
                        for c in range(EMB // 16):
                            vrow[pl.ds(c * 16, 16)] = vrow[pl.ds(c * 16, 16)] * SCALE

            pltpu.emit_pipeline(
                body,
                grid=(n_tok // WINDOW,),
                in_specs=[pl.BlockSpec((2, n_cols), index_map=lambda i: (i, 0))],
                out_specs=[pl.BlockSpec((WINDOW, EMB), index_map=lambda i: (i, 0))],
                core_axis_name=("core", "subcore"),
                dimension_semantics=(pltpu.PARALLEL,),
            )(i_hbm, o_hbm)

        return k(table, indices)

    out = run(embedding_weight, idx)
    return out.reshape(tokens.shape + (EMB,))


# speedup vs baseline: 1.2597x; 1.2597x over previous
import math

import jax
import jax.numpy as jnp
from jax import lax
from jax.experimental import pallas as pl
from jax.experimental.pallas import tpu as pltpu
from jax.experimental.pallas import tpu_sc as plsc

EMB = 64
SCALE = math.sqrt(EMB)
TOK_PER_WIN = 200
WPG = 16
NW = 32
TOK_PER_W = 819200 // NW
WINS_PER_W = TOK_PER_W // TOK_PER_WIN
GROUPS = WINS_PER_W // WPG
GROUP_ROWS = WPG * TOK_PER_WIN // 128
K1_BLOCK = 4000


def _window_pieces(q):
    lo, hi = q * TOK_PER_WIN, (q + 1) * TOK_PER_WIN
    bounds = [lo] + [b for b in range((lo // 128 + 1) * 128, hi, 128)] + [hi]
    return [(a, b - a) for a, b in zip(bounds[:-1], bounds[1:])]


def _scale_dup(table):
    vocab = table.shape[0]

    def body(x_ref, o_ref):
        x = x_ref[...] * SCALE
        o_ref[...] = jnp.concatenate([x, x], axis=-1)

    return pl.pallas_call(
        body,
        grid=(vocab // K1_BLOCK,),
        in_specs=[pl.BlockSpec((K1_BLOCK, EMB), lambda i: (i, 0))],
        out_specs=pl.BlockSpec((K1_BLOCK, 2 * EMB), lambda i: (i, 0)),
        out_shape=jax.ShapeDtypeStruct((vocab, 2 * EMB), jnp.float32),
    )(table)


def kernel(tokens, embedding_weight):
    n_rows, n_cols = tokens.shape
    n_tok = n_rows * n_cols
    idx = tokens.reshape(n_tok // 128, 128).astype(jnp.int32)

    mesh = plsc.VectorSubcoreMesh(core_axis_name="core", subcore_axis_name="subcore")

    @jax.jit
    def run(table, indices):
        tabled = _scale_dup(table)

        @pl.kernel(
            out_type=jax.ShapeDtypeStruct((n_rows, n_cols, EMB), jnp.float32),
            mesh=mesh,
            scratch_types=[
                pltpu.VMEM((TOK_PER_W // 128, 128), jnp.int32),
                pltpu.VMEM((128, 2 * EMB), jnp.float32),
                pltpu.VMEM((128, 2 * EMB), jnp.float32),
                pltpu.VMEM((TOK_PER_WIN, EMB), jnp.float32),
                pltpu.VMEM((TOK_PER_WIN, EMB), jnp.float32),
                pltpu.SemaphoreType.DMA,
                pltpu.SemaphoreType.DMA((2,)),
                pltpu.SemaphoreType.DMA((2,)),
            ],
        )
        def k(x_hbm, i_hbm, o_hbm, idx_v, g_v0, g_v1, o_v0, o_v1, isem, gsems, osems):
            o64 = o_hbm.reshape(n_tok, EMB)
            wid = lax.axis_index("subcore") * 2 + lax.axis_index("core")
            irow0 = wid * (TOK_PER_W // 128)

            for r8 in range(0, TOK_PER_W // 128, 8):
                pltpu.async_copy(
                    i_hbm.at[pl.ds(irow0 + r8, 8)],
                    idx_v.at[pl.ds(r8, 8)],
                    isem,
                ).wait()

            def select(cnt, off, gv, o_v):
                @pl.loop(0, cnt)
                def _(r):
                    src = gv.at[r]
                    dst = o_v.at[off + r]
                    for c in range(EMB // 16):
                        dst[pl.ds(c * 16, 16)] = src[pl.ds(c * 16, 16)]

            @pl.loop(0, GROUPS)
            def _(grp):
                grow0 = grp * GROUP_ROWS
                inflight = []

                def fire(flat, cnt, j, o_v, off):
                    gv = g_v0 if j % 2 == 0 else g_v1
                    if len(inflight) >= 2:
                        cp, cnt_p, off_p, gv_p, ov_p = inflight.pop(0)
                        cp.wait()
                        select(cnt_p, off_p, gv_p, ov_p)
                    cp = pltpu.async_copy(
                        x_hbm.at[idx_v.at[grow0 + flat // 128,
                                          pl.ds(flat % 128, cnt)]],
                        gv.at[pl.ds(0, cnt)],
                        gsems.at[j % 2],
                    )
                    inflight.append((cp, cnt, off, gv, o_v))

                def drain():
                    while inflight:
                        cp, cnt_p, off_p, gv_p, ov_p = inflight.pop(0)
                        cp.wait()
                        select(cnt_p, off_p, gv_p, ov_p)

                j = 0
                for q in range(WPG):
                    o_v = o_v0 if q % 2 == 0 else o_v1
                    osem = osems.at[q % 2]
                    win = grp * WPG + q
                    tok0 = wid * TOK_PER_W + win * TOK_PER_WIN

                    if q >= 2:
                        pltpu.make_async_copy(
                            o_v, o64.at[pl.ds(tok0, TOK_PER_WIN)], osem
                        ).wait()
                    else:
                        @pl.when(grp > 0)
                        def _():
                            pltpu.make_async_copy(
                                o_v, o64.at[pl.ds(tok0, TOK_PER_WIN)], osem
                            ).wait()

                    for flat, cnt in _window_pieces(q):
                        fire(flat, cnt, j, o_v, flat - q * TOK_PER_WIN)
                        j += 1
                    drain()

                    pltpu.async_copy(o_v, o64.at[pl.ds(tok0, TOK_PER_WIN)], osem)

            for b in range(2):
                pltpu.make_async_copy(
                    o_v0 if b == 0 else o_v1,
                    o64.at[pl.ds(0, TOK_PER_WIN)],
                    osems.at[b],
                ).wait()

        return k(tabled, indices)

    return run(embedding_weight, idx)


# speedup vs baseline: 1.4838x; 1.1779x over previous
import math

import jax
import jax.numpy as jnp
from jax import lax
from jax.experimental import pallas as pl
from jax.experimental.pallas import tpu as pltpu
from jax.experimental.pallas import tpu_sc as plsc

EMB = 64
SCALE = math.sqrt(EMB)
TOK_PER_WIN = 200
WPG = 16
NW = 32
TOK_PER_W = 819200 // NW
WINS_PER_W = TOK_PER_W // TOK_PER_WIN
GROUPS = WINS_PER_W // WPG
GROUP_ROWS = WPG * TOK_PER_WIN // 128
K1_BLOCK = 2048


def _window_pieces(q):
    lo, hi = q * TOK_PER_WIN, (q + 1) * TOK_PER_WIN
    bounds = [lo] + [b for b in range((lo // 128 + 1) * 128, hi, 128)] + [hi]
    return [(a, b - a) for a, b in zip(bounds[:-1], bounds[1:])]


def _scale_dup(table_t):
    vocab = table_t.shape[1]

    def body(x_ref, o_ref):
        x = x_ref[...].T * SCALE
        o_ref[...] = jnp.concatenate([x, x], axis=-1)

    grid = (vocab + K1_BLOCK - 1) // K1_BLOCK
    return pl.pallas_call(
        body,
        grid=(grid,),
        in_specs=[pl.BlockSpec((EMB, K1_BLOCK), lambda i: (0, i))],
        out_specs=pl.BlockSpec((K1_BLOCK, 2 * EMB), lambda i: (i, 0)),
        out_shape=jax.ShapeDtypeStruct((vocab, 2 * EMB), jnp.float32),
    )(table_t)


def kernel(tokens, embedding_weight):
    n_rows, n_cols = tokens.shape
    n_tok = n_rows * n_cols
    idx = tokens.reshape(n_tok // 128, 128).astype(jnp.int32)

    mesh = plsc.VectorSubcoreMesh(core_axis_name="core", subcore_axis_name="subcore")

    @jax.jit
    def run(table, indices):
        tabled = _scale_dup(table.T)

        @pl.kernel(
            out_type=jax.ShapeDtypeStruct((n_rows, n_cols, EMB), jnp.float32),
            mesh=mesh,
            scratch_types=[
                pltpu.VMEM((TOK_PER_W // 128, 128), jnp.int32),
                pltpu.VMEM((128, 2 * EMB), jnp.float32),
                pltpu.VMEM((128, 2 * EMB), jnp.float32),
                pltpu.VMEM((TOK_PER_WIN, EMB), jnp.float32),
                pltpu.VMEM((TOK_PER_WIN, EMB), jnp.float32),
                pltpu.SemaphoreType.DMA,
                pltpu.SemaphoreType.DMA((2,)),
                pltpu.SemaphoreType.DMA((2,)),
            ],
        )
        def k(x_hbm, i_hbm, o_hbm, idx_v, g_v0, g_v1, o_v0, o_v1, isem, gsems, osems):
            o64 = o_hbm.reshape(n_tok, EMB)
            wid = lax.axis_index("subcore") * 2 + lax.axis_index("core")
            irow0 = wid * (TOK_PER_W // 128)

            for r8 in range(0, TOK_PER_W // 128, 8):
                pltpu.async_copy(
                    i_hbm.at[pl.ds(irow0 + r8, 8)],
                    idx_v.at[pl.ds(r8, 8)],
                    isem,
                ).wait()

            def select(cnt, off, gv, o_v):
                @pl.loop(0, cnt)
                def _(r):
                    src = gv.at[r]
                    dst = o_v.at[off + r]
                    for c in range(EMB // 16):
                        dst[pl.ds(c * 16, 16)] = src[pl.ds(c * 16, 16)]

            @pl.loop(0, GROUPS)
            def _(grp):
                grow0 = grp * GROUP_ROWS
                inflight = []

                def fire(flat, cnt, j, o_v, off):
                    gv = g_v0 if j % 2 == 0 else g_v1
                    if len(inflight) >= 2:
                        cp, cnt_p, off_p, gv_p, ov_p = inflight.pop(0)
                        cp.wait()
                        select(cnt_p, off_p, gv_p, ov_p)
                    cp = pltpu.async_copy(
                        x_hbm.at[idx_v.at[grow0 + flat // 128,
                                          pl.ds(flat % 128, cnt)]],
                        gv.at[pl.ds(0, cnt)],
                        gsems.at[j % 2],
                    )
                    inflight.append((cp, cnt, off, gv, o_v))

                def drain():
                    while inflight:
                        cp, cnt_p, off_p, gv_p, ov_p = inflight.pop(0)
                        cp.wait()
                        select(cnt_p, off_p, gv_p, ov_p)

                j = 0
                for q in range(WPG):
                    o_v = o_v0 if q % 2 == 0 else o_v1
                    osem = osems.at[q % 2]
                    win = grp * WPG + q
                    tok0 = wid * TOK_PER_W + win * TOK_PER_WIN

                    if q >= 2:
                        pltpu.make_async_copy(
                            o_v, o64.at[pl.ds(tok0, TOK_PER_WIN)], osem
                        ).wait()
                    else:
                        @pl.when(grp > 0)
                        def _():
                            pltpu.make_async_copy(
                                o_v, o64.at[pl.ds(tok0, TOK_PER_WIN)], osem
                            ).wait()

                    for flat, cnt in _window_pieces(q):
                        fire(flat, cnt, j, o_v, flat - q * TOK_PER_WIN)
                        j += 1
                    drain()

                    pltpu.async_copy(o_v, o64.at[pl.ds(tok0, TOK_PER_WIN)], osem)

            for b in range(2):
                pltpu.make_async_copy(
                    o_v0 if b == 0 else o_v1,
                    o64.at[pl.ds(0, TOK_PER_WIN)],
                    osems.at[b],
                ).wait()

        return k(tabled, indices)

    return run(embedding_weight, idx)
